# baseline (speedup 1.0000x reference)
import functools

import jax
import jax.numpy as jnp
from jax import lax
from jax.experimental import pallas as pl
from jax.experimental.pallas import tpu as pltpu
from jax.experimental.pallas import tpu_sc as plsc

N = 10000
E = 320000
ET = E + N
DIN = 128
DH = 128
Q = 32
G = 16

NP = 10240
BN = 512
NB = NP // BN

NC = 2
NS = 16
NW = NC * NS
C = 32
P = 10368
TP = NW * P
TPA = TP + 2 * C
RPT = NP // NS


NBUF = 3
NCH = P // C


def _edge_body(src_hbm, dst_hbm, as_hbm, ad_hbm, h_hbm,
               acc_out, den_out,
               asv, adv, denv, srcv, dstv, wv, rows, acc_sh, *gsem):
    cid = lax.axis_index("c")
    sid = lax.axis_index("s")
    wid = sid * NC + cid

    pltpu.sync_copy(as_hbm, asv)
    pltpu.sync_copy(ad_hbm, adv)

    zf = jnp.zeros((16,), jnp.float32)

    def _zden(i, carry):
        denv[pl.ds(pl.multiple_of(i * 16, 16), 16)] = zf
        return carry

    lax.fori_loop(0, NP // 16, _zden, 0)

    def _zrows(r, carry):
        for k in range(8):
            rows[0, r, pl.ds(k * 16, 16)] = zf
        return carry

    lax.fori_loop(0, C, _zrows, 0)

    r0 = sid * RPT
    for b in range(RPT // C):
        pltpu.sync_copy(rows.at[0], acc_sh.at[pl.ds(r0 + b * C, C), :])
    plsc.subcore_barrier()

    def _prep(ci, b):
        base = wid * P + ci * C
        pltpu.sync_copy(src_hbm.at[pl.ds(base, C)], srcv.at[b])
        pltpu.sync_copy(dst_hbm.at[pl.ds(base, C)], dstv.at[b])
        pltpu.async_copy(h_hbm.at[srcv.at[b]], rows.at[b], gsem[b])

    for b in range(NBUF - 1):
        _prep(b, b)

    def _outer(co, carry):
        for b0 in range(NBUF):
            ci = co * NBUF + b0
            bn = (b0 + NBUF - 1) % NBUF
            base = wid * P + ci * C

            _prep(ci + NBUF - 1, bn)

            for g in range(C // 16):
                sv = srcv[b0, pl.ds(g * 16, 16)]
                dv = dstv[b0, pl.ds(g * 16, 16)]
                e = plsc.load_gather(asv, [sv]) + plsc.load_gather(adv, [dv])
                e = jnp.where(e >= 0.0, e, 0.2 * e)
                w = jnp.exp(e)
                eid = base + g * 16 + lax.iota(jnp.int32, 16)
                w = jnp.where(eid < ET, w, 0.0)
                wv[b0, pl.ds(g * 16, 16)] = w
                plsc.addupdate_scatter(denv, [dv], w)

            pltpu.make_async_copy(
                h_hbm.at[srcv.at[b0]], rows.at[b0], gsem[b0]).wait()
            for el in range(C):
                ws = plsc.load_gather(wv.at[b0], [jnp.full((16,), el, jnp.int32)])
                for k in range(8):
                    rows[b0, el, pl.ds(k * 16, 16)] = (
                        rows[b0, el, pl.ds(k * 16, 16)] * ws)
            pltpu.sync_copy(rows.at[b0], acc_sh.at[dstv.at[b0]], add=True)
        return carry

    lax.fori_loop(0, NCH // NBUF, _outer, 0)
    for ce in (NCH, NCH + 1):
        be = ce % NBUF
        pltpu.make_async_copy(
            h_hbm.at[srcv.at[be]], rows.at[be], gsem[be]).wait()
    plsc.subcore_barrier()

    for b in range(RPT // C):
        pltpu.sync_copy(acc_sh.at[pl.ds(r0 + b * C, C), :],
                        acc_out.at[cid, pl.ds(r0 + b * C, C), :])
    pltpu.sync_copy(denv, den_out.at[wid])


@functools.cache
def _edge_pass_kernel():
    return pl.kernel(
        _edge_body,
        out_type=(jax.ShapeDtypeStruct((NC, NP, DH), jnp.float32),
                  jax.ShapeDtypeStruct((NW, NP), jnp.float32)),
        mesh=plsc.VectorSubcoreMesh(core_axis_name="c", subcore_axis_name="s",
                                    num_cores=NC, num_subcores=NS),
        compiler_params=pltpu.CompilerParams(needs_layout_passes=False),
        scratch_types=(
        pltpu.VMEM((NP,), jnp.float32),
        pltpu.VMEM((NP,), jnp.float32),
        pltpu.VMEM((NP,), jnp.float32),
        pltpu.VMEM((NBUF, C), jnp.int32),
        pltpu.VMEM((NBUF, C), jnp.int32),
        pltpu.VMEM((NBUF, C), jnp.float32),
        pltpu.VMEM((NBUF, C, DH), jnp.float32),
        pltpu.VMEM_SHARED((NP, DH), jnp.float32),
        pltpu.SemaphoreType.DMA,
        pltpu.SemaphoreType.DMA,
        pltpu.SemaphoreType.DMA,
        ),
    )


def _edge_pass(src, dst, a_s, a_d, h):
    return _edge_pass_kernel()(src, dst, a_s, a_d, h)


def _node_first_body(x_ref, w_ref, a2_ref, h_ref, alp_ref):
    h = jnp.dot(x_ref[:], w_ref[:], preferred_element_type=jnp.float32)
    h_ref[:] = h
    alp_ref[:] = lax.dot_general(a2_ref[:], h, (((0,), (1,)), ((), ())),
                                 preferred_element_type=jnp.float32)


def _node_first(x, w, a2):
    return pl.pallas_call(
        _node_first_body,
        grid=(NB,),
        in_specs=[
            pl.BlockSpec((BN, DIN), lambda i: (i, 0)),
            pl.BlockSpec((DIN, DH), lambda i: (0, 0)),
            pl.BlockSpec((DH, 8), lambda i: (0, 0)),
        ],
        out_specs=[
            pl.BlockSpec((BN, DH), lambda i: (i, 0)),
            pl.BlockSpec((8, BN), lambda i: (0, i)),
        ],
        out_shape=[
            jax.ShapeDtypeStruct((NP, DH), jnp.float32),
            jax.ShapeDtypeStruct((8, NP), jnp.float32),
        ],
    )(x, w, a2)


def _finish(acc_ref, den_ref, b_ref):
    num = acc_ref[0] + acc_ref[1]
    den = jnp.maximum(jnp.sum(den_ref[:], axis=0), 1e-30)[:, None]
    return jnp.maximum(num / den + b_ref[:][0:1, :], 0.0)


def _node_mid_body(acc_ref, den_ref, b_ref, w_ref, a2_ref, h_ref, alp_ref):
    hin = _finish(acc_ref, den_ref, b_ref)
    h = jnp.dot(hin, w_ref[:], preferred_element_type=jnp.float32)
    h_ref[:] = h
    alp_ref[:] = lax.dot_general(a2_ref[:], h, (((0,), (1,)), ((), ())),
                                 preferred_element_type=jnp.float32)


def _node_mid(acc, den, b8, w, a2):
    return pl.pallas_call(
        _node_mid_body,
        grid=(NB,),
        in_specs=[
            pl.BlockSpec((NC, BN, DH), lambda i: (0, i, 0)),
            pl.BlockSpec((NW, BN), lambda i: (0, i)),
            pl.BlockSpec((8, DH), lambda i: (0, 0)),
            pl.BlockSpec((DH, DH), lambda i: (0, 0)),
            pl.BlockSpec((DH, 8), lambda i: (0, 0)),
        ],
        out_specs=[
            pl.BlockSpec((BN, DH), lambda i: (i, 0)),
            pl.BlockSpec((8, BN), lambda i: (0, i)),
        ],
        out_shape=[
            jax.ShapeDtypeStruct((NP, DH), jnp.float32),
            jax.ShapeDtypeStruct((8, NP), jnp.float32),
        ],
    )(acc, den, b8, w, a2)


def _qkv_body(acc_ref, den_ref, b_ref, inw_ref, inb_ref, q_ref, k_ref, v_ref):
    hin = _finish(acc_ref, den_ref, b_ref)
    qkv = jnp.dot(hin, inw_ref[:], preferred_element_type=jnp.float32)
    qkv = qkv + inb_ref[:][0:1, :]
    q_ref[:] = qkv[:, :DH]
    k_ref[:] = qkv[:, DH:2 * DH]
    v_ref[:] = qkv[:, 2 * DH:]


def _qkv(acc, den, b8, inw, inb8):
    return pl.pallas_call(
        _qkv_body,
        grid=(NB,),
        in_specs=[
            pl.BlockSpec((NC, BN, DH), lambda i: (0, i, 0)),
            pl.BlockSpec((NW, BN), lambda i: (0, i)),
            pl.BlockSpec((8, DH), lambda i: (0, 0)),
            pl.BlockSpec((DH, 3 * DH), lambda i: (0, 0)),
            pl.BlockSpec((8, 3 * DH), lambda i: (0, 0)),
        ],
        out_specs=[pl.BlockSpec((BN, DH), lambda i: (i, 0))] * 3,
        out_shape=[jax.ShapeDtypeStruct((NP, DH), jnp.float32)] * 3,
    )(acc, den, b8, inw, inb8)


def _ffn_body(qe_ref, w1_ref, b1_ref, w2_ref, b2_ref, inw_ref, inb_ref,
              tk_ref, tv_ref):
    t = jnp.dot(qe_ref[:], w1_ref[:], preferred_element_type=jnp.float32)
    t = jnp.maximum(t + b1_ref[:][0:1, :], 0.0)
    t = jnp.dot(t, w2_ref[:], preferred_element_type=jnp.float32)
    t = t + b2_ref[:][0:1, :]
    kv = jnp.dot(t, inw_ref[:][:, DH:], preferred_element_type=jnp.float32)
    kv = kv + inb_ref[:][0:1, DH:]
    tk_ref[:] = kv[:, :DH]
    tv_ref[:] = kv[:, DH:]


def _ffn(qe, w1, b18, w2, b28, inw, inb8):
    return pl.pallas_call(
        _ffn_body,
        out_shape=[jax.ShapeDtypeStruct((Q, DH), jnp.float32)] * 2,
    )(qe, w1, b18, w2, b28, inw, inb8)


def _attn_body(q_ref, k_ref, v_ref, ow_ref, ob_ref, cqw_ref, cqb_ref,
               tk_ref, tv_ref, cow_ref, cob_ref, out_ref, accs, dens):
    kj = pl.program_id(1)

    @pl.when(kj == 0)
    def _():
        accs[:] = jnp.zeros_like(accs)
        dens[:] = jnp.zeros_like(dens)

    logits = lax.dot_general(q_ref[:], k_ref[:], (((1,), (1,)), ((), ())),
                             preferred_element_type=jnp.float32)
    logits = logits * (1.0 / jnp.sqrt(jnp.float32(DH)))
    col = lax.broadcasted_iota(jnp.int32, (BN, BN), 1) + kj * BN
    s = jnp.where(col < N, jnp.exp(logits), 0.0)
    accs[:] += jnp.dot(s, v_ref[:], preferred_element_type=jnp.float32)
    dens[:] += jnp.sum(s, axis=1, keepdims=True)

    @pl.when(kj == pl.num_programs(1) - 1)
    def _():
        h2 = accs[:] / dens[:]
        h2 = jnp.dot(h2, ow_ref[:], preferred_element_type=jnp.float32)
        h2 = h2 + ob_ref[:][0:1, :]
        q2 = jnp.dot(h2, cqw_ref[:], preferred_element_type=jnp.float32)
        q2 = q2 + cqb_ref[:][0:1, :]
        l2 = lax.dot_general(q2, tk_ref[:], (((1,), (1,)), ((), ())),
                             preferred_element_type=jnp.float32)
        l2 = l2 * (1.0 / jnp.sqrt(jnp.float32(DH)))
        m = jnp.max(l2, axis=1, keepdims=True)
        p = jnp.exp(l2 - m)
        p = p / jnp.sum(p, axis=1, keepdims=True)
        h3 = jnp.dot(p, tv_ref[:], preferred_element_type=jnp.float32)
        h3 = jnp.dot(h3, cow_ref[:], preferred_element_type=jnp.float32)
        out_ref[:] = h3 + cob_ref[:][0:1, :]


def _attn(qp, kp, vp, ow, ob8, cqw, cqb8, tk, tv, cow, cob8):
    return pl.pallas_call(
        _attn_body,
        grid=(NB, NB),
        in_specs=[
            pl.BlockSpec((BN, DH), lambda qi, kj: (qi, 0)),
            pl.BlockSpec((BN, DH), lambda qi, kj: (kj, 0)),
            pl.BlockSpec((BN, DH), lambda qi, kj: (kj, 0)),
            pl.BlockSpec((DH, DH), lambda qi, kj: (0, 0)),
            pl.BlockSpec((8, DH), lambda qi, kj: (0, 0)),
            pl.BlockSpec((DH, DH), lambda qi, kj: (0, 0)),
            pl.BlockSpec((8, DH), lambda qi, kj: (0, 0)),
            pl.BlockSpec((Q, DH), lambda qi, kj: (0, 0)),
            pl.BlockSpec((Q, DH), lambda qi, kj: (0, 0)),
            pl.BlockSpec((DH, DH), lambda qi, kj: (0, 0)),
            pl.BlockSpec((8, DH), lambda qi, kj: (0, 0)),
        ],
        out_specs=pl.BlockSpec((BN, DH), lambda qi, kj: (qi, 0)),
        out_shape=jax.ShapeDtypeStruct((NP, DH), jnp.float32),
        scratch_shapes=[
            pltpu.VMEM((BN, DH), jnp.float32),
            pltpu.VMEM((BN, 1), jnp.float32),
        ],
    )(qp, kp, vp, ow, ob8, cqw, cqb8, tk, tv, cow, cob8)


def _pool_body(h_ref, b_ref, out_ref, sums, cnts):
    i = pl.program_id(0)

    @pl.when(i == 0)
    def _():
        sums[:] = jnp.zeros_like(sums)
        cnts[:] = jnp.zeros_like(cnts)

    onehot = (lax.broadcasted_iota(jnp.int32, (G, BN), 0)
              == b_ref[0]).astype(jnp.float32)
    sums[:] += jnp.dot(onehot, h_ref[:], preferred_element_type=jnp.float32)
    cnts[:] += jnp.sum(onehot, axis=1, keepdims=True)

    @pl.when(i == pl.num_programs(0) - 1)
    def _():
        out_ref[:] = sums[:] / jnp.maximum(cnts[:], 1.0)


def _pool(h3, batch3):
    return pl.pallas_call(
        _pool_body,
        grid=(NB,),
        in_specs=[
            pl.BlockSpec((BN, DH), lambda i: (i, 0)),
            pl.BlockSpec((1, 1, BN), lambda i: (i, 0, 0)),
        ],
        out_specs=pl.BlockSpec((G, DH), lambda i: (0, 0)),
        out_shape=jax.ShapeDtypeStruct((G, DH), jnp.float32),
        scratch_shapes=[
            pltpu.VMEM((G, DH), jnp.float32),
            pltpu.VMEM((G, 1), jnp.float32),
        ],
    )(h3, batch3)


def _b8(v):
    return jnp.broadcast_to(v.reshape(1, -1), (8, v.shape[-1]))


def kernel(x, edge_index, batch, q_emb,
           gat0_W, gat0_as, gat0_ad, gat0_b,
           gat1_W, gat1_as, gat1_ad, gat1_b,
           gat2_W, gat2_as, gat2_ad, gat2_b,
           ffn_W1, ffn_b1, ffn_W2, ffn_b2,
           sa_in_w, sa_in_b, sa_out_w, sa_out_b,
           ca_in_w, ca_in_b, ca_out_w, ca_out_b):
    loop = jnp.arange(N, dtype=edge_index.dtype)
    src = jnp.concatenate([edge_index[0], loop])
    dst = jnp.concatenate([edge_index[1], loop])
    src = jnp.pad(src, (0, TPA - ET))
    dst = jnp.pad(dst, (0, TPA - ET))
    xp = jnp.pad(x, ((0, NP - N), (0, 0)))
    batch3 = jnp.pad(batch, (0, NP - N), constant_values=G).reshape(NB, 1, BN)

    a2s = [jnp.pad(jnp.stack([a_s, a_d], axis=1), ((0, 0), (0, 6)))
           for a_s, a_d in ((gat0_as, gat0_ad), (gat1_as, gat1_ad),
                            (gat2_as, gat2_ad))]

    h, alp = _node_first(xp, gat0_W, a2s[0])
    acc, den = _edge_pass(src, dst, alp[0], alp[1], h)
    h, alp = _node_mid(acc, den, _b8(gat0_b), gat1_W, a2s[1])
    acc, den = _edge_pass(src, dst, alp[0], alp[1], h)
    h, alp = _node_mid(acc, den, _b8(gat1_b), gat2_W, a2s[2])
    acc, den = _edge_pass(src, dst, alp[0], alp[1], h)

    qp, kp, vp = _qkv(acc, den, _b8(gat2_b), sa_in_w, _b8(sa_in_b))
    tk, tv = _ffn(q_emb, ffn_W1, _b8(ffn_b1), ffn_W2, _b8(ffn_b2),
                  ca_in_w, _b8(ca_in_b))
    h3 = _attn(qp, kp, vp, sa_out_w, _b8(sa_out_b),
               ca_in_w[:, :DH], _b8(ca_in_b[:DH]), tk, tv,
               ca_out_w, _b8(ca_out_b))
    return _pool(h3, batch3)


# speedup vs baseline: 1.0717x; 1.0717x over previous
import functools

import jax
import jax.numpy as jnp
from jax import lax
from jax.experimental import pallas as pl
from jax.experimental.pallas import tpu as pltpu
from jax.experimental.pallas import tpu_sc as plsc

N = 10000
E = 320000
ET = E + N
DIN = 128
DH = 128
Q = 32
G = 16

NP = 10240
BN = 512
NB = NP // BN

NC = 2
NS = 16
NW = NC * NS
C = 128
P = 10368
TP = NW * P
TPA = TP + 2 * C
RPT = NP // NS


NBUF = 1
NCH = P // C


def _edge_body(src_hbm, dst_hbm, as_hbm, ad_hbm, h_hbm,
               acc_out, den_out,
               asv, adv, denv, srcv, dstv, wv, rows, acc_sh, gsem):
    cid = lax.axis_index("c")
    sid = lax.axis_index("s")
    wid = sid * NC + cid

    pltpu.sync_copy(as_hbm, asv)
    pltpu.sync_copy(ad_hbm, adv)

    zf = jnp.zeros((16,), jnp.float32)

    def _zden(i, carry):
        denv[pl.ds(pl.multiple_of(i * 16, 16), 16)] = zf
        return carry

    lax.fori_loop(0, NP // 16, _zden, 0)

    def _zrows(r, carry):
        for k in range(8):
            rows[0, r, pl.ds(k * 16, 16)] = zf
        return carry

    lax.fori_loop(0, C, _zrows, 0)

    r0 = sid * RPT
    for b in range(RPT // C):
        pltpu.sync_copy(rows.at[0], acc_sh.at[pl.ds(r0 + b * C, C), :])
    plsc.subcore_barrier()

    def _outer(ci, carry):
        base = wid * P + ci * C
        pltpu.sync_copy(src_hbm.at[pl.ds(base, C)], srcv.at[0])
        pltpu.sync_copy(dst_hbm.at[pl.ds(base, C)], dstv.at[0])
        pltpu.async_copy(h_hbm.at[srcv.at[0]], rows.at[0], gsem[0])

        for g in range(C // 16):
            sv = srcv[0, pl.ds(g * 16, 16)]
            dv = dstv[0, pl.ds(g * 16, 16)]
            e = plsc.load_gather(asv, [sv]) + plsc.load_gather(adv, [dv])
            e = jnp.where(e >= 0.0, e, 0.2 * e)
            w = jnp.exp(e)
            eid = base + g * 16 + lax.iota(jnp.int32, 16)
            w = jnp.where(eid < ET, w, 0.0)
            wv[0, pl.ds(g * 16, 16)] = w
            plsc.addupdate_scatter(denv, [dv], w)

        pltpu.make_async_copy(
            h_hbm.at[srcv.at[0]], rows.at[0], gsem[0]).wait()
        for el in range(C):
            ws = plsc.load_gather(wv.at[0], [jnp.full((16,), el, jnp.int32)])
            for k in range(8):
                rows[0, el, pl.ds(k * 16, 16)] = (
                    rows[0, el, pl.ds(k * 16, 16)] * ws)
        pltpu.sync_copy(rows.at[0], acc_sh.at[dstv.at[0]], add=True)
        return carry

    lax.fori_loop(0, NCH, _outer, 0)
    plsc.subcore_barrier()

    for b in range(RPT // C):
        pltpu.sync_copy(acc_sh.at[pl.ds(r0 + b * C, C), :],
                        acc_out.at[cid, pl.ds(r0 + b * C, C), :])
    pltpu.sync_copy(denv, den_out.at[wid])


@functools.cache
def _edge_pass_kernel():
    return pl.kernel(
        _edge_body,
        out_type=(jax.ShapeDtypeStruct((NC, NP, DH), jnp.float32),
                  jax.ShapeDtypeStruct((NW, NP), jnp.float32)),
        mesh=plsc.VectorSubcoreMesh(core_axis_name="c", subcore_axis_name="s",
                                    num_cores=NC, num_subcores=NS),
        compiler_params=pltpu.CompilerParams(needs_layout_passes=False),
        scratch_types=(
        pltpu.VMEM((NP,), jnp.float32),
        pltpu.VMEM((NP,), jnp.float32),
        pltpu.VMEM((NP,), jnp.float32),
        pltpu.VMEM((NBUF, C), jnp.int32),
        pltpu.VMEM((NBUF, C), jnp.int32),
        pltpu.VMEM((NBUF, C), jnp.float32),
        pltpu.VMEM((NBUF, C, DH), jnp.float32),
        pltpu.VMEM_SHARED((NP, DH), jnp.float32),
        (pltpu.SemaphoreType.DMA,) * NBUF,
        ),
    )


def _edge_pass(src, dst, a_s, a_d, h):
    return _edge_pass_kernel()(src, dst, a_s, a_d, h)


def _node_first_body(x_ref, w_ref, a2_ref, h_ref, alp_ref):
    h = jnp.dot(x_ref[:], w_ref[:], preferred_element_type=jnp.float32)
    h_ref[:] = h
    alp_ref[:] = lax.dot_general(a2_ref[:], h, (((0,), (1,)), ((), ())),
                                 preferred_element_type=jnp.float32)


def _node_first(x, w, a2):
    return pl.pallas_call(
        _node_first_body,
        grid=(NB,),
        in_specs=[
            pl.BlockSpec((BN, DIN), lambda i: (i, 0)),
            pl.BlockSpec((DIN, DH), lambda i: (0, 0)),
            pl.BlockSpec((DH, 8), lambda i: (0, 0)),
        ],
        out_specs=[
            pl.BlockSpec((BN, DH), lambda i: (i, 0)),
            pl.BlockSpec((8, BN), lambda i: (0, i)),
        ],
        out_shape=[
            jax.ShapeDtypeStruct((NP, DH), jnp.float32),
            jax.ShapeDtypeStruct((8, NP), jnp.float32),
        ],
    )(x, w, a2)


def _finish(acc_ref, den_ref, b_ref):
    num = acc_ref[0] + acc_ref[1]
    den = jnp.maximum(jnp.sum(den_ref[:], axis=0), 1e-30)[:, None]
    return jnp.maximum(num / den + b_ref[:][0:1, :], 0.0)


def _node_mid_body(acc_ref, den_ref, b_ref, w_ref, a2_ref, h_ref, alp_ref):
    hin = _finish(acc_ref, den_ref, b_ref)
    h = jnp.dot(hin, w_ref[:], preferred_element_type=jnp.float32)
    h_ref[:] = h
    alp_ref[:] = lax.dot_general(a2_ref[:], h, (((0,), (1,)), ((), ())),
                                 preferred_element_type=jnp.float32)


def _node_mid(acc, den, b8, w, a2):
    return pl.pallas_call(
        _node_mid_body,
        grid=(NB,),
        in_specs=[
            pl.BlockSpec((NC, BN, DH), lambda i: (0, i, 0)),
            pl.BlockSpec((NW, BN), lambda i: (0, i)),
            pl.BlockSpec((8, DH), lambda i: (0, 0)),
            pl.BlockSpec((DH, DH), lambda i: (0, 0)),
            pl.BlockSpec((DH, 8), lambda i: (0, 0)),
        ],
        out_specs=[
            pl.BlockSpec((BN, DH), lambda i: (i, 0)),
            pl.BlockSpec((8, BN), lambda i: (0, i)),
        ],
        out_shape=[
            jax.ShapeDtypeStruct((NP, DH), jnp.float32),
            jax.ShapeDtypeStruct((8, NP), jnp.float32),
        ],
    )(acc, den, b8, w, a2)


def _qkv_body(acc_ref, den_ref, b_ref, inw_ref, inb_ref, q_ref, k_ref, v_ref):
    hin = _finish(acc_ref, den_ref, b_ref)
    qkv = jnp.dot(hin, inw_ref[:], preferred_element_type=jnp.float32)
    qkv = qkv + inb_ref[:][0:1, :]
    q_ref[:] = qkv[:, :DH]
    k_ref[:] = qkv[:, DH:2 * DH]
    v_ref[:] = qkv[:, 2 * DH:]


def _qkv(acc, den, b8, inw, inb8):
    return pl.pallas_call(
        _qkv_body,
        grid=(NB,),
        in_specs=[
            pl.BlockSpec((NC, BN, DH), lambda i: (0, i, 0)),
            pl.BlockSpec((NW, BN), lambda i: (0, i)),
            pl.BlockSpec((8, DH), lambda i: (0, 0)),
            pl.BlockSpec((DH, 3 * DH), lambda i: (0, 0)),
            pl.BlockSpec((8, 3 * DH), lambda i: (0, 0)),
        ],
        out_specs=[pl.BlockSpec((BN, DH), lambda i: (i, 0))] * 3,
        out_shape=[jax.ShapeDtypeStruct((NP, DH), jnp.float32)] * 3,
    )(acc, den, b8, inw, inb8)


def _ffn_body(qe_ref, w1_ref, b1_ref, w2_ref, b2_ref, inw_ref, inb_ref,
              tk_ref, tv_ref):
    t = jnp.dot(qe_ref[:], w1_ref[:], preferred_element_type=jnp.float32)
    t = jnp.maximum(t + b1_ref[:][0:1, :], 0.0)
    t = jnp.dot(t, w2_ref[:], preferred_element_type=jnp.float32)
    t = t + b2_ref[:][0:1, :]
    kv = jnp.dot(t, inw_ref[:][:, DH:], preferred_element_type=jnp.float32)
    kv = kv + inb_ref[:][0:1, DH:]
    tk_ref[:] = kv[:, :DH]
    tv_ref[:] = kv[:, DH:]


def _ffn(qe, w1, b18, w2, b28, inw, inb8):
    return pl.pallas_call(
        _ffn_body,
        out_shape=[jax.ShapeDtypeStruct((Q, DH), jnp.float32)] * 2,
    )(qe, w1, b18, w2, b28, inw, inb8)


def _attn_body(q_ref, k_ref, v_ref, ow_ref, ob_ref, cqw_ref, cqb_ref,
               tk_ref, tv_ref, cow_ref, cob_ref, out_ref, accs, dens):
    kj = pl.program_id(1)

    @pl.when(kj == 0)
    def _():
        accs[:] = jnp.zeros_like(accs)
        dens[:] = jnp.zeros_like(dens)

    logits = lax.dot_general(q_ref[:], k_ref[:], (((1,), (1,)), ((), ())),
                             preferred_element_type=jnp.float32)
    logits = logits * (1.0 / jnp.sqrt(jnp.float32(DH)))
    col = lax.broadcasted_iota(jnp.int32, (BN, BN), 1) + kj * BN
    s = jnp.where(col < N, jnp.exp(logits), 0.0)
    accs[:] += jnp.dot(s, v_ref[:], preferred_element_type=jnp.float32)
    dens[:] += jnp.sum(s, axis=1, keepdims=True)

    @pl.when(kj == pl.num_programs(1) - 1)
    def _():
        h2 = accs[:] / dens[:]
        h2 = jnp.dot(h2, ow_ref[:], preferred_element_type=jnp.float32)
        h2 = h2 + ob_ref[:][0:1, :]
        q2 = jnp.dot(h2, cqw_ref[:], preferred_element_type=jnp.float32)
        q2 = q2 + cqb_ref[:][0:1, :]
        l2 = lax.dot_general(q2, tk_ref[:], (((1,), (1,)), ((), ())),
                             preferred_element_type=jnp.float32)
        l2 = l2 * (1.0 / jnp.sqrt(jnp.float32(DH)))
        m = jnp.max(l2, axis=1, keepdims=True)
        p = jnp.exp(l2 - m)
        p = p / jnp.sum(p, axis=1, keepdims=True)
        h3 = jnp.dot(p, tv_ref[:], preferred_element_type=jnp.float32)
        h3 = jnp.dot(h3, cow_ref[:], preferred_element_type=jnp.float32)
        out_ref[:] = h3 + cob_ref[:][0:1, :]


def _attn(qp, kp, vp, ow, ob8, cqw, cqb8, tk, tv, cow, cob8):
    return pl.pallas_call(
        _attn_body,
        grid=(NB, NB),
        in_specs=[
            pl.BlockSpec((BN, DH), lambda qi, kj: (qi, 0)),
            pl.BlockSpec((BN, DH), lambda qi, kj: (kj, 0)),
            pl.BlockSpec((BN, DH), lambda qi, kj: (kj, 0)),
            pl.BlockSpec((DH, DH), lambda qi, kj: (0, 0)),
            pl.BlockSpec((8, DH), lambda qi, kj: (0, 0)),
            pl.BlockSpec((DH, DH), lambda qi, kj: (0, 0)),
            pl.BlockSpec((8, DH), lambda qi, kj: (0, 0)),
            pl.BlockSpec((Q, DH), lambda qi, kj: (0, 0)),
            pl.BlockSpec((Q, DH), lambda qi, kj: (0, 0)),
            pl.BlockSpec((DH, DH), lambda qi, kj: (0, 0)),
            pl.BlockSpec((8, DH), lambda qi, kj: (0, 0)),
        ],
        out_specs=pl.BlockSpec((BN, DH), lambda qi, kj: (qi, 0)),
        out_shape=jax.ShapeDtypeStruct((NP, DH), jnp.float32),
        scratch_shapes=[
            pltpu.VMEM((BN, DH), jnp.float32),
            pltpu.VMEM((BN, 1), jnp.float32),
        ],
    )(qp, kp, vp, ow, ob8, cqw, cqb8, tk, tv, cow, cob8)


def _pool_body(h_ref, b_ref, out_ref, sums, cnts):
    i = pl.program_id(0)

    @pl.when(i == 0)
    def _():
        sums[:] = jnp.zeros_like(sums)
        cnts[:] = jnp.zeros_like(cnts)

    onehot = (lax.broadcasted_iota(jnp.int32, (G, BN), 0)
              == b_ref[0]).astype(jnp.float32)
    sums[:] += jnp.dot(onehot, h_ref[:], preferred_element_type=jnp.float32)
    cnts[:] += jnp.sum(onehot, axis=1, keepdims=True)

    @pl.when(i == pl.num_programs(0) - 1)
    def _():
        out_ref[:] = sums[:] / jnp.maximum(cnts[:], 1.0)


def _pool(h3, batch3):
    return pl.pallas_call(
        _pool_body,
        grid=(NB,),
        in_specs=[
            pl.BlockSpec((BN, DH), lambda i: (i, 0)),
            pl.BlockSpec((1, 1, BN), lambda i: (i, 0, 0)),
        ],
        out_specs=pl.BlockSpec((G, DH), lambda i: (0, 0)),
        out_shape=jax.ShapeDtypeStruct((G, DH), jnp.float32),
        scratch_shapes=[
            pltpu.VMEM((G, DH), jnp.float32),
            pltpu.VMEM((G, 1), jnp.float32),
        ],
    )(h3, batch3)


def _b8(v):
    return jnp.broadcast_to(v.reshape(1, -1), (8, v.shape[-1]))


def kernel(x, edge_index, batch, q_emb,
           gat0_W, gat0_as, gat0_ad, gat0_b,
           gat1_W, gat1_as, gat1_ad, gat1_b,
           gat2_W, gat2_as, gat2_ad, gat2_b,
           ffn_W1, ffn_b1, ffn_W2, ffn_b2,
           sa_in_w, sa_in_b, sa_out_w, sa_out_b,
           ca_in_w, ca_in_b, ca_out_w, ca_out_b):
    loop = jnp.arange(N, dtype=edge_index.dtype)
    src = jnp.concatenate([edge_index[0], loop])
    dst = jnp.concatenate([edge_index[1], loop])
    src = jnp.pad(src, (0, TPA - ET))
    dst = jnp.pad(dst, (0, TPA - ET))
    xp = jnp.pad(x, ((0, NP - N), (0, 0)))
    batch3 = jnp.pad(batch, (0, NP - N), constant_values=G).reshape(NB, 1, BN)

    a2s = [jnp.pad(jnp.stack([a_s, a_d], axis=1), ((0, 0), (0, 6)))
           for a_s, a_d in ((gat0_as, gat0_ad), (gat1_as, gat1_ad),
                            (gat2_as, gat2_ad))]

    h, alp = _node_first(xp, gat0_W, a2s[0])
    acc, den = _edge_pass(src, dst, alp[0], alp[1], h)
    h, alp = _node_mid(acc, den, _b8(gat0_b), gat1_W, a2s[1])
    acc, den = _edge_pass(src, dst, alp[0], alp[1], h)
    h, alp = _node_mid(acc, den, _b8(gat1_b), gat2_W, a2s[2])
    acc, den = _edge_pass(src, dst, alp[0], alp[1], h)

    qp, kp, vp = _qkv(acc, den, _b8(gat2_b), sa_in_w, _b8(sa_in_b))
    tk, tv = _ffn(q_emb, ffn_W1, _b8(ffn_b1), ffn_W2, _b8(ffn_b2),
                  ca_in_w, _b8(ca_in_b))
    h3 = _attn(qp, kp, vp, sa_out_w, _b8(sa_out_b),
               ca_in_w[:, :DH], _b8(ca_in_b[:DH]), tk, tv,
               ca_out_w, _b8(ca_out_b))
    return _pool(h3, batch3)


# speedup vs baseline: 1.6564x; 1.5455x over previous
import functools

import jax
import jax.numpy as jnp
from jax import lax
from jax.experimental import pallas as pl
from jax.experimental.pallas import tpu as pltpu
from jax.experimental.pallas import tpu_sc as plsc

N = 10000
E = 320000
ET = E + N
DIN = 128
DH = 128
Q = 32
G = 16

NP = 10240
BN = 512
NB = NP // BN

NC = 2
NS = 16
NW = NC * NS
C = 128
P = 10368
TP = NW * P
TPA = TP + 2 * C
RPT = NP // NS


CE = 64
NCH = P // CE
NIB = 3
UNR = 6


def _edge_body(src_hbm, dst_hbm, as_hbm, ad_hbm, h_hbm,
               acc_out, den_out,
               asv, adv, denv, srcv, dstv, wv, rows, acc_sh,
               gsem, ssem, isem):
    cid = lax.axis_index("c")
    sid = lax.axis_index("s")
    wid = sid * NC + cid

    pltpu.sync_copy(as_hbm, asv)
    pltpu.sync_copy(ad_hbm, adv)

    zf = jnp.zeros((16,), jnp.float32)

    def _zden(i, carry):
        denv[pl.ds(pl.multiple_of(i * 16, 16), 16)] = zf
        return carry

    lax.fori_loop(0, NP // 16, _zden, 0)

    def _zrows(r, carry):
        for k in range(8):
            rows[0, r, pl.ds(k * 16, 16)] = zf
        return carry

    lax.fori_loop(0, CE, _zrows, 0)

    r0 = sid * RPT
    for b in range(RPT // CE):
        pltpu.sync_copy(rows.at[0], acc_sh.at[pl.ds(r0 + b * CE, CE), :])
    plsc.subcore_barrier()

    def _idx_prep(ci, s):
        base = wid * P + ci * CE
        pltpu.async_copy(src_hbm.at[pl.ds(base, CE)], srcv.at[s], isem[s])
        pltpu.async_copy(dst_hbm.at[pl.ds(base, CE)], dstv.at[s], isem[s])

    def _idx_wait(ci, s):
        base = wid * P + ci * CE
        pltpu.make_async_copy(src_hbm.at[pl.ds(base, CE)], srcv.at[s],
                              isem[s]).wait()
        pltpu.make_async_copy(dst_hbm.at[pl.ds(base, CE)], dstv.at[s],
                              isem[s]).wait()

    _idx_prep(0, 0)
    _idx_prep(1, 1)
    _idx_wait(0, 0)
    pltpu.async_copy(h_hbm.at[srcv.at[0]], rows.at[0], gsem[0])

    def _chunk(ci, j):
        b0 = j % 2
        b1 = (j + 1) % 2
        i0 = j % NIB
        i1 = (j + 1) % NIB
        i2 = (j + 2) % NIB
        base = wid * P + ci * CE

        @pl.when(ci >= 1)
        def _():
            pltpu.make_async_copy(rows.at[b1], acc_sh.at[dstv.at[i2]],
                                  ssem[b1]).wait()

        _idx_wait(ci + 1, i1)
        pltpu.async_copy(h_hbm.at[srcv.at[i1]], rows.at[b1], gsem[b1])
        _idx_prep(ci + 2, i2)

        for g in range(CE // 16):
            sv = srcv[i0, pl.ds(g * 16, 16)]
            dv = dstv[i0, pl.ds(g * 16, 16)]
            e = plsc.load_gather(asv, [sv]) + plsc.load_gather(adv, [dv])
            e = jnp.where(e >= 0.0, e, 0.2 * e)
            w = jnp.exp(e)
            eid = base + g * 16 + lax.iota(jnp.int32, 16)
            w = jnp.where(eid < ET, w, 0.0)
            wv[pl.ds(g * 16, 16)] = w
            plsc.addupdate_scatter(denv, [dv], w)

        pltpu.make_async_copy(h_hbm.at[srcv.at[i0]], rows.at[b0],
                              gsem[b0]).wait()

        def _scale(g8, carry):
            for t in range(8):
                el = g8 * 8 + t
                ws = plsc.load_gather(wv, [jnp.full((16,), el, jnp.int32)])
                for k in range(8):
                    rows[b0, el, pl.ds(k * 16, 16)] = (
                        rows[b0, el, pl.ds(k * 16, 16)] * ws)
            return carry

        lax.fori_loop(0, CE // 8, _scale, 0)
        pltpu.async_copy(rows.at[b0], acc_sh.at[dstv.at[i0]], ssem[b0],
                         add=True)

    def _outer(it, carry):
        for j in range(UNR):
            _chunk(it * UNR + j, j)
        return carry

    lax.fori_loop(0, NCH // UNR, _outer, 0)
    pltpu.make_async_copy(rows.at[(NCH - 1) % 2],
                          acc_sh.at[dstv.at[(NCH - 1) % NIB]],
                          ssem[(NCH - 1) % 2]).wait()
    pltpu.make_async_copy(h_hbm.at[srcv.at[NCH % NIB]],
                          rows.at[NCH % 2], gsem[NCH % 2]).wait()
    _idx_wait(NCH + 1, (NCH + 1) % NIB)
    plsc.subcore_barrier()

    for b in range(RPT // CE):
        pltpu.sync_copy(acc_sh.at[pl.ds(r0 + b * CE, CE), :],
                        acc_out.at[cid, pl.ds(r0 + b * CE, CE), :])
    pltpu.sync_copy(denv, den_out.at[wid])


@functools.cache
def _edge_pass_kernel():
    return pl.kernel(
        _edge_body,
        out_type=(jax.ShapeDtypeStruct((NC, NP, DH), jnp.float32),
                  jax.ShapeDtypeStruct((NW, NP), jnp.float32)),
        mesh=plsc.VectorSubcoreMesh(core_axis_name="c", subcore_axis_name="s",
                                    num_cores=NC, num_subcores=NS),
        compiler_params=pltpu.CompilerParams(needs_layout_passes=False),
        scratch_types=(
        pltpu.VMEM((NP,), jnp.float32),
        pltpu.VMEM((NP,), jnp.float32),
        pltpu.VMEM((NP,), jnp.float32),
        pltpu.VMEM((NIB, CE), jnp.int32),
        pltpu.VMEM((NIB, CE), jnp.int32),
        pltpu.VMEM((CE,), jnp.float32),
        pltpu.VMEM((2, CE, DH), jnp.float32),
        pltpu.VMEM_SHARED((NP, DH), jnp.float32),
        (pltpu.SemaphoreType.DMA,) * 2,
        (pltpu.SemaphoreType.DMA,) * 2,
        (pltpu.SemaphoreType.DMA,) * NIB,
        ),
    )


def _edge_pass(src, dst, a_s, a_d, h):
    return _edge_pass_kernel()(src, dst, a_s, a_d, h)


def _node_first_body(x_ref, w_ref, a2_ref, h_ref, alp_ref):
    h = jnp.dot(x_ref[:], w_ref[:], preferred_element_type=jnp.float32)
    h_ref[:] = h
    alp_ref[:] = lax.dot_general(a2_ref[:], h, (((0,), (1,)), ((), ())),
                                 preferred_element_type=jnp.float32)


def _node_first(x, w, a2):
    return pl.pallas_call(
        _node_first_body,
        grid=(NB,),
        in_specs=[
            pl.BlockSpec((BN, DIN), lambda i: (i, 0)),
            pl.BlockSpec((DIN, DH), lambda i: (0, 0)),
            pl.BlockSpec((DH, 8), lambda i: (0, 0)),
        ],
        out_specs=[
            pl.BlockSpec((BN, DH), lambda i: (i, 0)),
            pl.BlockSpec((8, BN), lambda i: (0, i)),
        ],
        out_shape=[
            jax.ShapeDtypeStruct((NP, DH), jnp.float32),
            jax.ShapeDtypeStruct((8, NP), jnp.float32),
        ],
    )(x, w, a2)


def _finish(acc_ref, den_ref, b_ref):
    num = acc_ref[0] + acc_ref[1]
    den = jnp.maximum(jnp.sum(den_ref[:], axis=0), 1e-30)[:, None]
    return jnp.maximum(num / den + b_ref[:][0:1, :], 0.0)


def _node_mid_body(acc_ref, den_ref, b_ref, w_ref, a2_ref, h_ref, alp_ref):
    hin = _finish(acc_ref, den_ref, b_ref)
    h = jnp.dot(hin, w_ref[:], preferred_element_type=jnp.float32)
    h_ref[:] = h
    alp_ref[:] = lax.dot_general(a2_ref[:], h, (((0,), (1,)), ((), ())),
                                 preferred_element_type=jnp.float32)


def _node_mid(acc, den, b8, w, a2):
    return pl.pallas_call(
        _node_mid_body,
        grid=(NB,),
        in_specs=[
            pl.BlockSpec((NC, BN, DH), lambda i: (0, i, 0)),
            pl.BlockSpec((NW, BN), lambda i: (0, i)),
            pl.BlockSpec((8, DH), lambda i: (0, 0)),
            pl.BlockSpec((DH, DH), lambda i: (0, 0)),
            pl.BlockSpec((DH, 8), lambda i: (0, 0)),
        ],
        out_specs=[
            pl.BlockSpec((BN, DH), lambda i: (i, 0)),
            pl.BlockSpec((8, BN), lambda i: (0, i)),
        ],
        out_shape=[
            jax.ShapeDtypeStruct((NP, DH), jnp.float32),
            jax.ShapeDtypeStruct((8, NP), jnp.float32),
        ],
    )(acc, den, b8, w, a2)


def _qkv_body(acc_ref, den_ref, b_ref, inw_ref, inb_ref, q_ref, k_ref, v_ref):
    hin = _finish(acc_ref, den_ref, b_ref)
    qkv = jnp.dot(hin, inw_ref[:], preferred_element_type=jnp.float32)
    qkv = qkv + inb_ref[:][0:1, :]
    q_ref[:] = qkv[:, :DH]
    k_ref[:] = qkv[:, DH:2 * DH]
    v_ref[:] = qkv[:, 2 * DH:]


def _qkv(acc, den, b8, inw, inb8):
    return pl.pallas_call(
        _qkv_body,
        grid=(NB,),
        in_specs=[
            pl.BlockSpec((NC, BN, DH), lambda i: (0, i, 0)),
            pl.BlockSpec((NW, BN), lambda i: (0, i)),
            pl.BlockSpec((8, DH), lambda i: (0, 0)),
            pl.BlockSpec((DH, 3 * DH), lambda i: (0, 0)),
            pl.BlockSpec((8, 3 * DH), lambda i: (0, 0)),
        ],
        out_specs=[pl.BlockSpec((BN, DH), lambda i: (i, 0))] * 3,
        out_shape=[jax.ShapeDtypeStruct((NP, DH), jnp.float32)] * 3,
    )(acc, den, b8, inw, inb8)


def _ffn_body(qe_ref, w1_ref, b1_ref, w2_ref, b2_ref, inw_ref, inb_ref,
              tk_ref, tv_ref):
    t = jnp.dot(qe_ref[:], w1_ref[:], preferred_element_type=jnp.float32)
    t = jnp.maximum(t + b1_ref[:][0:1, :], 0.0)
    t = jnp.dot(t, w2_ref[:], preferred_element_type=jnp.float32)
    t = t + b2_ref[:][0:1, :]
    kv = jnp.dot(t, inw_ref[:][:, DH:], preferred_element_type=jnp.float32)
    kv = kv + inb_ref[:][0:1, DH:]
    tk_ref[:] = kv[:, :DH]
    tv_ref[:] = kv[:, DH:]


def _ffn(qe, w1, b18, w2, b28, inw, inb8):
    return pl.pallas_call(
        _ffn_body,
        out_shape=[jax.ShapeDtypeStruct((Q, DH), jnp.float32)] * 2,
    )(qe, w1, b18, w2, b28, inw, inb8)


def _attn_body(q_ref, k_ref, v_ref, ow_ref, ob_ref, cqw_ref, cqb_ref,
               tk_ref, tv_ref, cow_ref, cob_ref, out_ref, accs, dens):
    kj = pl.program_id(1)

    @pl.when(kj == 0)
    def _():
        accs[:] = jnp.zeros_like(accs)
        dens[:] = jnp.zeros_like(dens)

    logits = lax.dot_general(q_ref[:], k_ref[:], (((1,), (1,)), ((), ())),
                             preferred_element_type=jnp.float32)
    logits = logits * (1.0 / jnp.sqrt(jnp.float32(DH)))
    col = lax.broadcasted_iota(jnp.int32, (BN, BN), 1) + kj * BN
    s = jnp.where(col < N, jnp.exp(logits), 0.0)
    accs[:] += jnp.dot(s, v_ref[:], preferred_element_type=jnp.float32)
    dens[:] += jnp.sum(s, axis=1, keepdims=True)

    @pl.when(kj == pl.num_programs(1) - 1)
    def _():
        h2 = accs[:] / dens[:]
        h2 = jnp.dot(h2, ow_ref[:], preferred_element_type=jnp.float32)
        h2 = h2 + ob_ref[:][0:1, :]
        q2 = jnp.dot(h2, cqw_ref[:], preferred_element_type=jnp.float32)
        q2 = q2 + cqb_ref[:][0:1, :]
        l2 = lax.dot_general(q2, tk_ref[:], (((1,), (1,)), ((), ())),
                             preferred_element_type=jnp.float32)
        l2 = l2 * (1.0 / jnp.sqrt(jnp.float32(DH)))
        m = jnp.max(l2, axis=1, keepdims=True)
        p = jnp.exp(l2 - m)
        p = p / jnp.sum(p, axis=1, keepdims=True)
        h3 = jnp.dot(p, tv_ref[:], preferred_element_type=jnp.float32)
        h3 = jnp.dot(h3, cow_ref[:], preferred_element_type=jnp.float32)
        out_ref[:] = h3 + cob_ref[:][0:1, :]


def _attn(qp, kp, vp, ow, ob8, cqw, cqb8, tk, tv, cow, cob8):
    return pl.pallas_call(
        _attn_body,
        grid=(NB, NB),
        in_specs=[
            pl.BlockSpec((BN, DH), lambda qi, kj: (qi, 0)),
            pl.BlockSpec((BN, DH), lambda qi, kj: (kj, 0)),
            pl.BlockSpec((BN, DH), lambda qi, kj: (kj, 0)),
            pl.BlockSpec((DH, DH), lambda qi, kj: (0, 0)),
            pl.BlockSpec((8, DH), lambda qi, kj: (0, 0)),
            pl.BlockSpec((DH, DH), lambda qi, kj: (0, 0)),
            pl.BlockSpec((8, DH), lambda qi, kj: (0, 0)),
            pl.BlockSpec((Q, DH), lambda qi, kj: (0, 0)),
            pl.BlockSpec((Q, DH), lambda qi, kj: (0, 0)),
            pl.BlockSpec((DH, DH), lambda qi, kj: (0, 0)),
            pl.BlockSpec((8, DH), lambda qi, kj: (0, 0)),
        ],
        out_specs=pl.BlockSpec((BN, DH), lambda qi, kj: (qi, 0)),
        out_shape=jax.ShapeDtypeStruct((NP, DH), jnp.float32),
        scratch_shapes=[
            pltpu.VMEM((BN, DH), jnp.float32),
            pltpu.VMEM((BN, 1), jnp.float32),
        ],
    )(qp, kp, vp, ow, ob8, cqw, cqb8, tk, tv, cow, cob8)


def _pool_body(h_ref, b_ref, out_ref, sums, cnts):
    i = pl.program_id(0)

    @pl.when(i == 0)
    def _():
        sums[:] = jnp.zeros_like(sums)
        cnts[:] = jnp.zeros_like(cnts)

    onehot = (lax.broadcasted_iota(jnp.int32, (G, BN), 0)
              == b_ref[0]).astype(jnp.float32)
    sums[:] += jnp.dot(onehot, h_ref[:], preferred_element_type=jnp.float32)
    cnts[:] += jnp.sum(onehot, axis=1, keepdims=True)

    @pl.when(i == pl.num_programs(0) - 1)
    def _():
        out_ref[:] = sums[:] / jnp.maximum(cnts[:], 1.0)


def _pool(h3, batch3):
    return pl.pallas_call(
        _pool_body,
        grid=(NB,),
        in_specs=[
            pl.BlockSpec((BN, DH), lambda i: (i, 0)),
            pl.BlockSpec((1, 1, BN), lambda i: (i, 0, 0)),
        ],
        out_specs=pl.BlockSpec((G, DH), lambda i: (0, 0)),
        out_shape=jax.ShapeDtypeStruct((G, DH), jnp.float32),
        scratch_shapes=[
            pltpu.VMEM((G, DH), jnp.float32),
            pltpu.VMEM((G, 1), jnp.float32),
        ],
    )(h3, batch3)


def _b8(v):
    return jnp.broadcast_to(v.reshape(1, -1), (8, v.shape[-1]))


def kernel(x, edge_index, batch, q_emb,
           gat0_W, gat0_as, gat0_ad, gat0_b,
           gat1_W, gat1_as, gat1_ad, gat1_b,
           gat2_W, gat2_as, gat2_ad, gat2_b,
           ffn_W1, ffn_b1, ffn_W2, ffn_b2,
           sa_in_w, sa_in_b, sa_out_w, sa_out_b,
           ca_in_w, ca_in_b, ca_out_w, ca_out_b):
    loop = jnp.arange(N, dtype=edge_index.dtype)
    src = jnp.concatenate([edge_index[0], loop])
    dst = jnp.concatenate([edge_index[1], loop])
    src = jnp.pad(src, (0, TPA - ET))
    dst = jnp.pad(dst, (0, TPA - ET))
    xp = jnp.pad(x, ((0, NP - N), (0, 0)))
    batch3 = jnp.pad(batch, (0, NP - N), constant_values=G).reshape(NB, 1, BN)

    a2s = [jnp.pad(jnp.stack([a_s, a_d], axis=1), ((0, 0), (0, 6)))
           for a_s, a_d in ((gat0_as, gat0_ad), (gat1_as, gat1_ad),
                            (gat2_as, gat2_ad))]

    h, alp = _node_first(xp, gat0_W, a2s[0])
    acc, den = _edge_pass(src, dst, alp[0], alp[1], h)
    h, alp = _node_mid(acc, den, _b8(gat0_b), gat1_W, a2s[1])
    acc, den = _edge_pass(src, dst, alp[0], alp[1], h)
    h, alp = _node_mid(acc, den, _b8(gat1_b), gat2_W, a2s[2])
    acc, den = _edge_pass(src, dst, alp[0], alp[1], h)

    qp, kp, vp = _qkv(acc, den, _b8(gat2_b), sa_in_w, _b8(sa_in_b))
    tk, tv = _ffn(q_emb, ffn_W1, _b8(ffn_b1), ffn_W2, _b8(ffn_b2),
                  ca_in_w, _b8(ca_in_b))
    h3 = _attn(qp, kp, vp, sa_out_w, _b8(sa_out_b),
               ca_in_w[:, :DH], _b8(ca_in_b[:DH]), tk, tv,
               ca_out_w, _b8(ca_out_b))
    return _pool(h3, batch3)


# speedup vs baseline: 1.6661x; 1.0058x over previous
import functools

import jax
import jax.numpy as jnp
from jax import lax
from jax.experimental import pallas as pl
from jax.experimental.pallas import tpu as pltpu
from jax.experimental.pallas import tpu_sc as plsc

N = 10000
E = 320000
ET = E + N
DIN = 128
DH = 128
Q = 32
G = 16

NP = 10240
BN = 512
NB = NP // BN

NC = 2
NS = 16
NW = NC * NS
C = 128
P = 10368
TP = NW * P
TPA = TP + 2 * C
RPT = NP // NS


CE = 64
NCH = P // CE
NIB = 3
UNR = 6


def _edge_body(src_hbm, dst_hbm, as_hbm, ad_hbm, h_hbm,
               acc_out, den_out,
               asv, adv, denv, srcv, dstv, wv, rows, acc_sh,
               gsem, ssem, isem):
    cid = lax.axis_index("c")
    sid = lax.axis_index("s")
    wid = sid * NC + cid

    pltpu.sync_copy(as_hbm, asv)
    pltpu.sync_copy(ad_hbm, adv)

    zf = jnp.zeros((16,), jnp.float32)

    def _zden(i, carry):
        denv[pl.ds(pl.multiple_of(i * 16, 16), 16)] = zf
        return carry

    lax.fori_loop(0, NP // 16, _zden, 0)

    def _zrows(r, carry):
        for k in range(8):
            rows[0, r, pl.ds(k * 16, 16)] = zf
        return carry

    lax.fori_loop(0, CE, _zrows, 0)

    r0 = sid * RPT
    for b in range(RPT // CE):
        pltpu.sync_copy(rows.at[0], acc_sh.at[pl.ds(r0 + b * CE, CE), :])
    plsc.subcore_barrier()

    def _idx_prep(ci, s):
        base = wid * P + ci * CE
        pltpu.async_copy(src_hbm.at[pl.ds(base, CE)], srcv.at[s], isem[s])
        pltpu.async_copy(dst_hbm.at[pl.ds(base, CE)], dstv.at[s], isem[s])

    def _idx_wait(ci, s):
        base = wid * P + ci * CE
        pltpu.make_async_copy(src_hbm.at[pl.ds(base, CE)], srcv.at[s],
                              isem[s]).wait()
        pltpu.make_async_copy(dst_hbm.at[pl.ds(base, CE)], dstv.at[s],
                              isem[s]).wait()

    _idx_prep(0, 0)
    _idx_prep(1, 1)
    _idx_wait(0, 0)
    pltpu.async_copy(h_hbm.at[srcv.at[0]], rows.at[0], gsem[0])

    def _chunk(ci, j):
        b0 = j % 2
        b1 = (j + 1) % 2
        i0 = j % NIB
        i1 = (j + 1) % NIB
        i2 = (j + 2) % NIB
        base = wid * P + ci * CE

        @pl.when(ci >= 1)
        def _():
            pltpu.make_async_copy(rows.at[b1], acc_sh.at[dstv.at[i2]],
                                  ssem[b1]).wait()

        _idx_wait(ci + 1, i1)
        pltpu.async_copy(h_hbm.at[srcv.at[i1]], rows.at[b1], gsem[b1])
        _idx_prep(ci + 2, i2)

        for g in range(CE // 16):
            sv = srcv[i0, pl.ds(g * 16, 16)]
            dv = dstv[i0, pl.ds(g * 16, 16)]
            e = plsc.load_gather(asv, [sv]) + plsc.load_gather(adv, [dv])
            e = jnp.where(e >= 0.0, e, 0.2 * e)
            w = jnp.exp(e)
            eid = base + g * 16 + lax.iota(jnp.int32, 16)
            w = jnp.where(eid < ET, w, 0.0)
            wv[pl.ds(g * 16, 16)] = w
            plsc.addupdate_scatter(denv, [dv], w)

        pltpu.make_async_copy(h_hbm.at[srcv.at[i0]], rows.at[b0],
                              gsem[b0]).wait()

        def _scale(g8, carry):
            for t in range(8):
                el = g8 * 8 + t
                ws = plsc.load_gather(wv, [jnp.full((16,), el, jnp.int32)])
                for k in range(8):
                    rows[b0, el, pl.ds(k * 16, 16)] = (
                        rows[b0, el, pl.ds(k * 16, 16)] * ws)
            return carry

        lax.fori_loop(0, CE // 8, _scale, 0)
        pltpu.async_copy(rows.at[b0], acc_sh.at[dstv.at[i0]], ssem[b0],
                         add=True)

    def _outer(it, carry):
        for j in range(UNR):
            _chunk(it * UNR + j, j)
        return carry

    lax.fori_loop(0, NCH // UNR, _outer, 0)
    pltpu.make_async_copy(rows.at[(NCH - 1) % 2],
                          acc_sh.at[dstv.at[(NCH - 1) % NIB]],
                          ssem[(NCH - 1) % 2]).wait()
    pltpu.make_async_copy(h_hbm.at[srcv.at[NCH % NIB]],
                          rows.at[NCH % 2], gsem[NCH % 2]).wait()
    _idx_wait(NCH + 1, (NCH + 1) % NIB)
    plsc.subcore_barrier()

    for b in range(RPT // CE):
        pltpu.sync_copy(acc_sh.at[pl.ds(r0 + b * CE, CE), :],
                        acc_out.at[cid, pl.ds(r0 + b * CE, CE), :])
    pltpu.sync_copy(denv, den_out.at[wid])


@functools.cache
def _edge_pass_kernel():
    return pl.kernel(
        _edge_body,
        out_type=(jax.ShapeDtypeStruct((NC, NP, DH), jnp.float32),
                  jax.ShapeDtypeStruct((NW, NP), jnp.float32)),
        mesh=plsc.VectorSubcoreMesh(core_axis_name="c", subcore_axis_name="s",
                                    num_cores=NC, num_subcores=NS),
        compiler_params=pltpu.CompilerParams(needs_layout_passes=False),
        scratch_types=(
        pltpu.VMEM((NP,), jnp.float32),
        pltpu.VMEM((NP,), jnp.float32),
        pltpu.VMEM((NP,), jnp.float32),
        pltpu.VMEM((NIB, CE), jnp.int32),
        pltpu.VMEM((NIB, CE), jnp.int32),
        pltpu.VMEM((CE,), jnp.float32),
        pltpu.VMEM((2, CE, DH), jnp.float32),
        pltpu.VMEM_SHARED((NP, DH), jnp.float32),
        (pltpu.SemaphoreType.DMA,) * 2,
        (pltpu.SemaphoreType.DMA,) * 2,
        (pltpu.SemaphoreType.DMA,) * NIB,
        ),
    )


def _edge_pass(src, dst, a_s, a_d, h):
    return _edge_pass_kernel()(src, dst, a_s, a_d, h)


def _node_first_body(x_ref, w_ref, a2_ref, h_ref, alp_ref):
    h = jnp.dot(x_ref[:], w_ref[:], preferred_element_type=jnp.float32)
    h_ref[:] = h
    alp_ref[:] = lax.dot_general(a2_ref[:], h, (((0,), (1,)), ((), ())),
                                 preferred_element_type=jnp.float32)


def _node_first(x, w, a2):
    return pl.pallas_call(
        _node_first_body,
        grid=(NB,),
        in_specs=[
            pl.BlockSpec((BN, DIN), lambda i: (i, 0)),
            pl.BlockSpec((DIN, DH), lambda i: (0, 0)),
            pl.BlockSpec((DH, 8), lambda i: (0, 0)),
        ],
        out_specs=[
            pl.BlockSpec((BN, DH), lambda i: (i, 0)),
            pl.BlockSpec((8, BN), lambda i: (0, i)),
        ],
        out_shape=[
            jax.ShapeDtypeStruct((NP, DH), jnp.float32),
            jax.ShapeDtypeStruct((8, NP), jnp.float32),
        ],
    )(x, w, a2)


def _finish(acc_ref, den_ref, b_ref):
    num = acc_ref[0] + acc_ref[1]
    den = jnp.maximum(jnp.sum(den_ref[:], axis=0), 1e-30)[:, None]
    return jnp.maximum(num / den + b_ref[:][0:1, :], 0.0)


def _node_mid_body(acc_ref, den_ref, b_ref, w_ref, a2_ref, h_ref, alp_ref):
    hin = _finish(acc_ref, den_ref, b_ref)
    h = jnp.dot(hin, w_ref[:], preferred_element_type=jnp.float32)
    h_ref[:] = h
    alp_ref[:] = lax.dot_general(a2_ref[:], h, (((0,), (1,)), ((), ())),
                                 preferred_element_type=jnp.float32)


def _node_mid(acc, den, b8, w, a2):
    return pl.pallas_call(
        _node_mid_body,
        grid=(NB,),
        in_specs=[
            pl.BlockSpec((NC, BN, DH), lambda i: (0, i, 0)),
            pl.BlockSpec((NW, BN), lambda i: (0, i)),
            pl.BlockSpec((8, DH), lambda i: (0, 0)),
            pl.BlockSpec((DH, DH), lambda i: (0, 0)),
            pl.BlockSpec((DH, 8), lambda i: (0, 0)),
        ],
        out_specs=[
            pl.BlockSpec((BN, DH), lambda i: (i, 0)),
            pl.BlockSpec((8, BN), lambda i: (0, i)),
        ],
        out_shape=[
            jax.ShapeDtypeStruct((NP, DH), jnp.float32),
            jax.ShapeDtypeStruct((8, NP), jnp.float32),
        ],
    )(acc, den, b8, w, a2)


def _qkv_body(acc_ref, den_ref, b_ref, inw_ref, inb_ref, q_ref, k_ref, v_ref):
    hin = _finish(acc_ref, den_ref, b_ref)
    qkv = jnp.dot(hin, inw_ref[:], preferred_element_type=jnp.float32)
    qkv = qkv + inb_ref[:][0:1, :]
    q_ref[:] = qkv[:, :DH]
    k_ref[:] = qkv[:, DH:2 * DH]
    v_ref[:] = qkv[:, 2 * DH:]


def _qkv(acc, den, b8, inw, inb8):
    return pl.pallas_call(
        _qkv_body,
        grid=(NB,),
        in_specs=[
            pl.BlockSpec((NC, BN, DH), lambda i: (0, i, 0)),
            pl.BlockSpec((NW, BN), lambda i: (0, i)),
            pl.BlockSpec((8, DH), lambda i: (0, 0)),
            pl.BlockSpec((DH, 3 * DH), lambda i: (0, 0)),
            pl.BlockSpec((8, 3 * DH), lambda i: (0, 0)),
        ],
        out_specs=[pl.BlockSpec((BN, DH), lambda i: (i, 0))] * 3,
        out_shape=[jax.ShapeDtypeStruct((NP, DH), jnp.float32)] * 3,
    )(acc, den, b8, inw, inb8)


def _attn_body(q_ref, k_ref, v_ref, ow_ref, ob_ref, cqw_ref, cqb_ref,
               qe_ref, w1_ref, b1_ref, w2_ref, b2_ref, kvw_ref, kvb_ref,
               cow_ref, cob_ref, b_ref, out_ref,
               accs, dens, tks, tvs, sums, cnts):
    qi = pl.program_id(0)
    kj = pl.program_id(1)

    @pl.when((qi == 0) & (kj == 0))
    def _():
        t = jnp.dot(qe_ref[:], w1_ref[:], preferred_element_type=jnp.float32)
        t = jnp.maximum(t + b1_ref[:][0:1, :], 0.0)
        t = jnp.dot(t, w2_ref[:], preferred_element_type=jnp.float32)
        t = t + b2_ref[:][0:1, :]
        kv = jnp.dot(t, kvw_ref[:], preferred_element_type=jnp.float32)
        kv = kv + kvb_ref[:][0:1, :]
        tks[:] = kv[:, :DH]
        tvs[:] = kv[:, DH:]
        sums[:] = jnp.zeros_like(sums)
        cnts[:] = jnp.zeros_like(cnts)

    @pl.when(kj == 0)
    def _():
        accs[:] = jnp.zeros_like(accs)
        dens[:] = jnp.zeros_like(dens)

    logits = lax.dot_general(q_ref[:], k_ref[:], (((1,), (1,)), ((), ())),
                             preferred_element_type=jnp.float32)
    logits = logits * (1.0 / jnp.sqrt(jnp.float32(DH)))
    col = lax.broadcasted_iota(jnp.int32, (BN, BN), 1) + kj * BN
    s = jnp.where(col < N, jnp.exp(logits), 0.0)
    accs[:] += jnp.dot(s, v_ref[:], preferred_element_type=jnp.float32)
    dens[:] += jnp.sum(s, axis=1, keepdims=True)

    @pl.when(kj == pl.num_programs(1) - 1)
    def _():
        h2 = accs[:] / dens[:]
        h2 = jnp.dot(h2, ow_ref[:], preferred_element_type=jnp.float32)
        h2 = h2 + ob_ref[:][0:1, :]
        q2 = jnp.dot(h2, cqw_ref[:], preferred_element_type=jnp.float32)
        q2 = q2 + cqb_ref[:][0:1, :]
        l2 = lax.dot_general(q2, tks[:], (((1,), (1,)), ((), ())),
                             preferred_element_type=jnp.float32)
        l2 = l2 * (1.0 / jnp.sqrt(jnp.float32(DH)))
        m = jnp.max(l2, axis=1, keepdims=True)
        p = jnp.exp(l2 - m)
        p = p / jnp.sum(p, axis=1, keepdims=True)
        h3 = jnp.dot(p, tvs[:], preferred_element_type=jnp.float32)
        h3 = jnp.dot(h3, cow_ref[:], preferred_element_type=jnp.float32)
        h3 = h3 + cob_ref[:][0:1, :]
        onehot = (lax.broadcasted_iota(jnp.int32, (G, BN), 0)
                  == b_ref[0]).astype(jnp.float32)
        sums[:] += jnp.dot(onehot, h3, preferred_element_type=jnp.float32)
        cnts[:] += jnp.sum(onehot, axis=1, keepdims=True)

        @pl.when(qi == pl.num_programs(0) - 1)
        def _():
            out_ref[:] = sums[:] / jnp.maximum(cnts[:], 1.0)


def _attn(qp, kp, vp, ow, ob8, cqw, cqb8, qe, w1, b18, w2, b28, kvw, kvb8,
          cow, cob8, batch3):
    z = lambda qi, kj: (0, 0)
    return pl.pallas_call(
        _attn_body,
        grid=(NB, NB),
        in_specs=[
            pl.BlockSpec((BN, DH), lambda qi, kj: (qi, 0)),
            pl.BlockSpec((BN, DH), lambda qi, kj: (kj, 0)),
            pl.BlockSpec((BN, DH), lambda qi, kj: (kj, 0)),
            pl.BlockSpec((DH, DH), z),
            pl.BlockSpec((8, DH), z),
            pl.BlockSpec((DH, DH), z),
            pl.BlockSpec((8, DH), z),
            pl.BlockSpec((Q, 2 * DH), z),
            pl.BlockSpec((2 * DH, 2 * DH), z),
            pl.BlockSpec((8, 2 * DH), z),
            pl.BlockSpec((2 * DH, DH), z),
            pl.BlockSpec((8, DH), z),
            pl.BlockSpec((DH, 2 * DH), z),
            pl.BlockSpec((8, 2 * DH), z),
            pl.BlockSpec((DH, DH), z),
            pl.BlockSpec((8, DH), z),
            pl.BlockSpec((1, 1, BN), lambda qi, kj: (qi, 0, 0)),
        ],
        out_specs=pl.BlockSpec((G, DH), z),
        out_shape=jax.ShapeDtypeStruct((G, DH), jnp.float32),
        scratch_shapes=[
            pltpu.VMEM((BN, DH), jnp.float32),
            pltpu.VMEM((BN, 1), jnp.float32),
            pltpu.VMEM((Q, DH), jnp.float32),
            pltpu.VMEM((Q, DH), jnp.float32),
            pltpu.VMEM((G, DH), jnp.float32),
            pltpu.VMEM((G, 1), jnp.float32),
        ],
    )(qp, kp, vp, ow, ob8, cqw, cqb8, qe, w1, b18, w2, b28, kvw, kvb8,
      cow, cob8, batch3)


def _b8(v):
    return jnp.broadcast_to(v.reshape(1, -1), (8, v.shape[-1]))


def kernel(x, edge_index, batch, q_emb,
           gat0_W, gat0_as, gat0_ad, gat0_b,
           gat1_W, gat1_as, gat1_ad, gat1_b,
           gat2_W, gat2_as, gat2_ad, gat2_b,
           ffn_W1, ffn_b1, ffn_W2, ffn_b2,
           sa_in_w, sa_in_b, sa_out_w, sa_out_b,
           ca_in_w, ca_in_b, ca_out_w, ca_out_b):
    loop = jnp.arange(N, dtype=edge_index.dtype)
    src = jnp.concatenate([edge_index[0], loop])
    dst = jnp.concatenate([edge_index[1], loop])
    src = jnp.pad(src, (0, TPA - ET))
    dst = jnp.pad(dst, (0, TPA - ET))
    xp = jnp.pad(x, ((0, NP - N), (0, 0)))
    batch3 = jnp.pad(batch, (0, NP - N), constant_values=G).reshape(NB, 1, BN)

    a2s = [jnp.pad(jnp.stack([a_s, a_d], axis=1), ((0, 0), (0, 6)))
           for a_s, a_d in ((gat0_as, gat0_ad), (gat1_as, gat1_ad),
                            (gat2_as, gat2_ad))]

    h, alp = _node_first(xp, gat0_W, a2s[0])
    acc, den = _edge_pass(src, dst, alp[0], alp[1], h)
    h, alp = _node_mid(acc, den, _b8(gat0_b), gat1_W, a2s[1])
    acc, den = _edge_pass(src, dst, alp[0], alp[1], h)
    h, alp = _node_mid(acc, den, _b8(gat1_b), gat2_W, a2s[2])
    acc, den = _edge_pass(src, dst, alp[0], alp[1], h)

    qp, kp, vp = _qkv(acc, den, _b8(gat2_b), sa_in_w, _b8(sa_in_b))
    return _attn(qp, kp, vp, sa_out_w, _b8(sa_out_b),
                 ca_in_w[:, :DH], _b8(ca_in_b[:DH]),
                 q_emb, ffn_W1, _b8(ffn_b1), ffn_W2, _b8(ffn_b2),
                 ca_in_w[:, DH:], _b8(ca_in_b[DH:]),
                 ca_out_w, _b8(ca_out_b), batch3)
